# Initial kernel scaffold; baseline (speedup 1.0000x reference)
import jax
import jax.numpy as jnp
from jax.experimental import pallas as pl


def kernel(edge_energy, per_edge_scales, edge_index, atom_types):
    raise NotImplementedError("write your pallas kernel here")


# speedup vs baseline: 188.8054x; 188.8054x over previous
import math

import jax
import jax.numpy as jnp
from jax import lax
from jax.experimental import pallas as pl
from jax.experimental.pallas import tpu as pltpu
from jax.experimental.pallas import tpu_sc as plsc

N_NODES = 100000
N_EDGES = 6400000
NUM_TYPES = 16
FACTOR = 1.0 / math.sqrt(64.0)

NC = 2
NS = 16
LANES = 16

CHUNK = 2048
ROWS = CHUNK // 128
N_CHUNKS = N_EDGES // CHUNK
BASE_CHUNKS = N_CHUNKS // (NC * NS)
EXTRA_TILES = N_CHUNKS % (NC * NS)

N_PAD = 100352
SLICE = N_PAD // NS
ZCHUNK = SLICE // 4


def _sc_body(center_hbm, neigh_hbm, energy_hbm, species_hbm, table_hbm,
             part_hbm, species, table, cbuf, nbuf, ebuf, vbuf, zbuf, acc,
             sem):
    cid = lax.axis_index("c")
    sid = lax.axis_index("s")
    wid = sid * NC + cid

    pltpu.sync_copy(species_hbm, species)
    pltpu.sync_copy(table_hbm, table)

    zero = jnp.zeros((LANES,), jnp.float32)
    for i in range(ZCHUNK // LANES):
        zbuf[pl.ds(i * LANES, LANES)] = zero
    base = sid * SLICE
    for k in range(4):
        pltpu.sync_copy(zbuf, acc.at[pl.ds(base + k * ZCHUNK, ZCHUNK)])
    plsc.subcore_barrier()

    n_chunks = jnp.where(wid < EXTRA_TILES, BASE_CHUNKS + 1, BASE_CHUNKS)

    def chunk_body(g, _):
        ci = wid + g * (NC * NS)
        pltpu.sync_copy(center_hbm.at[ci], cbuf)
        pltpu.sync_copy(neigh_hbm.at[ci], nbuf)
        pltpu.sync_copy(energy_hbm.at[ci], ebuf)
        for j in range(CHUNK // LANES):
            r, o = j // 8, (j % 8) * LANES
            cvec = cbuf[r, pl.ds(o, LANES)]
            nvec = nbuf[pl.ds(j * LANES, LANES)]
            cs = plsc.load_gather(species, [cvec])
            ns = plsc.load_gather(species, [nvec])
            scale = plsc.load_gather(table, [cs * NUM_TYPES + ns])
            vbuf[r, pl.ds(o, LANES)] = ebuf[pl.ds(j * LANES, LANES)] * scale
        descs = [
            pltpu.async_copy(vbuf.at[s], acc.at[cbuf.at[s]], sem, add=True)
            for s in range(ROWS)
        ]
        for d in descs:
            d.wait()
        return _

    lax.fori_loop(0, n_chunks, chunk_body, None)

    plsc.subcore_barrier()
    pltpu.sync_copy(acc.at[pl.ds(base, SLICE)],
                    part_hbm.at[cid, pl.ds(base, SLICE)])


_sc_kernel = pl.kernel(
    _sc_body,
    out_type=jax.ShapeDtypeStruct((NC, N_PAD), jnp.float32),
    mesh=plsc.VectorSubcoreMesh(core_axis_name="c", subcore_axis_name="s"),
    compiler_params=pltpu.CompilerParams(needs_layout_passes=False),
    scratch_types=[
        pltpu.VMEM((N_NODES,), jnp.int32),
        pltpu.VMEM((NUM_TYPES * NUM_TYPES,), jnp.float32),
        pltpu.VMEM((ROWS, 128), jnp.int32),
        pltpu.VMEM((CHUNK,), jnp.int32),
        pltpu.VMEM((CHUNK,), jnp.float32),
        pltpu.VMEM((ROWS, 128), jnp.float32),
        pltpu.VMEM((ZCHUNK,), jnp.float32),
        pltpu.VMEM_SHARED((N_PAD,), jnp.float32),
        pltpu.SemaphoreType.DMA,
    ],
)


def _combine_body(p_ref, o_ref):
    o_ref[...] = (p_ref[0] + p_ref[1]) * FACTOR


_combine = pl.pallas_call(
    _combine_body,
    out_shape=jax.ShapeDtypeStruct((N_PAD // 128, 128), jnp.float32),
)


def kernel(edge_energy, per_edge_scales, edge_index, atom_types):
    center = edge_index[0].reshape(N_CHUNKS, ROWS, 128)
    neigh = edge_index[1].reshape(N_CHUNKS, CHUNK)
    energy = edge_energy.reshape(N_CHUNKS, CHUNK)
    species = atom_types.reshape(N_NODES)
    table = per_edge_scales.reshape(NUM_TYPES * NUM_TYPES)

    part = _sc_kernel(center, neigh, energy, species, table)
    out = _combine(part.reshape(NC, N_PAD // 128, 128))
    return out.reshape(N_PAD)[:N_NODES].reshape(N_NODES, 1)


# speedup vs baseline: 220.3490x; 1.1671x over previous
import math

import jax
import jax.numpy as jnp
from jax import lax
from jax.experimental import pallas as pl
from jax.experimental.pallas import tpu as pltpu
from jax.experimental.pallas import tpu_sc as plsc

N_NODES = 100000
N_EDGES = 6400000
NUM_TYPES = 16
FACTOR = 1.0 / math.sqrt(64.0)

NC = 2
NS = 16
LANES = 16

CHUNK = 2048
ROWS = CHUNK // 128
N_CHUNKS = N_EDGES // CHUNK
BASE_CHUNKS = N_CHUNKS // (NC * NS)
EXTRA_TILES = N_CHUNKS % (NC * NS)

N_PAD = 100352
SLICE = N_PAD // NS
ZCHUNK = SLICE // 4


N_PAIRS = (N_CHUNKS // (NC * NS) + 2) // 2


def _sc_body(center_hbm, neigh_hbm, energy_hbm, species_hbm, table_hbm,
             part_hbm, species, table, cbuf0, nbuf0, ebuf0, cbuf1, nbuf1,
             ebuf1, vbuf0, sidx0, vbuf1, sidx1, zbuf, acc, sem_in0, sem_in1,
             sem_sc0, sem_sc1):
    cid = lax.axis_index("c")
    sid = lax.axis_index("s")
    wid = sid * NC + cid
    cbuf = (cbuf0, cbuf1)
    nbuf = (nbuf0, nbuf1)
    ebuf = (ebuf0, ebuf1)
    vbuf = (vbuf0, vbuf1)
    sidx = (sidx0, sidx1)
    sem_in = (sem_in0, sem_in1)
    sem_sc = (sem_sc0, sem_sc1)

    def in_descs(k, b):
        ci = wid + k * (NC * NS)
        return [
            pltpu.make_async_copy(center_hbm.at[ci], cbuf[b], sem_in[b]),
            pltpu.make_async_copy(neigh_hbm.at[ci], nbuf[b], sem_in[b]),
            pltpu.make_async_copy(energy_hbm.at[ci], ebuf[b], sem_in[b]),
        ]

    def sc_descs(b):
        return [
            pltpu.make_async_copy(vbuf[b].at[s], acc.at[sidx[b].at[s]],
                                  sem_sc[b])
            for s in range(ROWS)
        ]

    def compute(b):
        for j in range(CHUNK // LANES):
            r, o = j // 8, (j % 8) * LANES
            cvec = cbuf[b][r, pl.ds(o, LANES)]
            nvec = nbuf[b][pl.ds(j * LANES, LANES)]
            cs = plsc.load_gather(species, [cvec])
            ns = plsc.load_gather(species, [nvec])
            scale = plsc.load_gather(table, [cs * NUM_TYPES + ns])
            vbuf[b][r, pl.ds(o, LANES)] = (
                ebuf[b][pl.ds(j * LANES, LANES)] * scale)
            sidx[b][r, pl.ds(o, LANES)] = cvec

    pltpu.sync_copy(species_hbm, species)
    pltpu.sync_copy(table_hbm, table)

    zero = jnp.zeros((LANES,), jnp.float32)
    for i in range(ZCHUNK // LANES):
        zbuf[pl.ds(i * LANES, LANES)] = zero
    base = sid * SLICE
    for k in range(4):
        pltpu.sync_copy(zbuf, acc.at[pl.ds(base + k * ZCHUNK, ZCHUNK)])
    plsc.subcore_barrier()

    n_chunks = jnp.where(wid < EXTRA_TILES, BASE_CHUNKS + 1, BASE_CHUNKS)

    for d in in_descs(0, 0):
        d.start()
    for d in in_descs(1, 1):
        d.start()

    def pair_body(p, _):
        for b in range(2):
            k = 2 * p + b

            @pl.when(k < n_chunks)
            def _process():
                for d in in_descs(k, b):
                    d.wait()

                @pl.when(k >= 2)
                def _drain():
                    for d in sc_descs(b):
                        d.wait()

                compute(b)
                for d in sc_descs(b):
                    d.start(add=True)

                @pl.when(k + 2 < n_chunks)
                def _prefetch():
                    for d in in_descs(k + 2, b):
                        d.start()

        return _

    lax.fori_loop(0, N_PAIRS, pair_body, None)

    for b in range(2):
        for d in sc_descs(b):
            d.wait()

    plsc.subcore_barrier()
    pltpu.sync_copy(acc.at[pl.ds(base, SLICE)],
                    part_hbm.at[cid, pl.ds(base, SLICE)])


_sc_kernel = pl.kernel(
    _sc_body,
    out_type=jax.ShapeDtypeStruct((NC, N_PAD), jnp.float32),
    mesh=plsc.VectorSubcoreMesh(core_axis_name="c", subcore_axis_name="s"),
    compiler_params=pltpu.CompilerParams(needs_layout_passes=False),
    scratch_types=[
        pltpu.VMEM((N_NODES,), jnp.int32),
        pltpu.VMEM((NUM_TYPES * NUM_TYPES,), jnp.float32),
        pltpu.VMEM((ROWS, 128), jnp.int32),
        pltpu.VMEM((CHUNK,), jnp.int32),
        pltpu.VMEM((CHUNK,), jnp.float32),
        pltpu.VMEM((ROWS, 128), jnp.int32),
        pltpu.VMEM((CHUNK,), jnp.int32),
        pltpu.VMEM((CHUNK,), jnp.float32),
        pltpu.VMEM((ROWS, 128), jnp.float32),
        pltpu.VMEM((ROWS, 128), jnp.int32),
        pltpu.VMEM((ROWS, 128), jnp.float32),
        pltpu.VMEM((ROWS, 128), jnp.int32),
        pltpu.VMEM((ZCHUNK,), jnp.float32),
        pltpu.VMEM_SHARED((N_PAD,), jnp.float32),
        pltpu.SemaphoreType.DMA,
        pltpu.SemaphoreType.DMA,
        pltpu.SemaphoreType.DMA,
        pltpu.SemaphoreType.DMA,
    ],
)


def _combine_body(p_ref, o_ref):
    o_ref[...] = (p_ref[0] + p_ref[1]) * FACTOR


_combine = pl.pallas_call(
    _combine_body,
    out_shape=jax.ShapeDtypeStruct((N_PAD // 128, 128), jnp.float32),
)


def kernel(edge_energy, per_edge_scales, edge_index, atom_types):
    center = edge_index[0].reshape(N_CHUNKS, ROWS, 128)
    neigh = edge_index[1].reshape(N_CHUNKS, CHUNK)
    energy = edge_energy.reshape(N_CHUNKS, CHUNK)
    species = atom_types.reshape(N_NODES)
    table = per_edge_scales.reshape(NUM_TYPES * NUM_TYPES)

    part = _sc_kernel(center, neigh, energy, species, table)
    out = _combine(part.reshape(NC, N_PAD // 128, 128))
    return out.reshape(N_PAD)[:N_NODES].reshape(N_NODES, 1)


# speedup vs baseline: 739.4032x; 3.3556x over previous
import math

import jax
import jax.numpy as jnp
from jax import lax
from jax.experimental import pallas as pl
from jax.experimental.pallas import tpu as pltpu
from jax.experimental.pallas import tpu_sc as plsc

N_NODES = 100000
N_EDGES = 6400000
NUM_TYPES = 16
FACTOR = 1.0 / math.sqrt(64.0)

NC = 2
NS = 16
LANES = 16

CHUNK = 2048
ROWS = CHUNK // 128
N_CHUNKS = N_EDGES // CHUNK
BASE_CHUNKS = N_CHUNKS // (NC * NS)
EXTRA_TILES = N_CHUNKS % (NC * NS)

N_PAD = 100352
SLICE = N_PAD // NS
ZCHUNK = SLICE // 4


N_PAIRS = (N_CHUNKS // (NC * NS) + 2) // 2


def _sc_body(edge_hbm, energy_hbm, species_hbm, table_hbm,
             part_hbm, species, table, cnbuf0, ebuf0, cnbuf1,
             ebuf1, vbuf0, sidx0, vbuf1, sidx1, zbuf, acc, sem_in0, sem_in1,
             sem_sc0, sem_sc1):
    cid = lax.axis_index("c")
    sid = lax.axis_index("s")
    wid = sid * NC + cid
    cnbuf = (cnbuf0, cnbuf1)
    ebuf = (ebuf0, ebuf1)
    vbuf = (vbuf0, vbuf1)
    sidx = (sidx0, sidx1)
    sem_in = (sem_in0, sem_in1)
    sem_sc = (sem_sc0, sem_sc1)

    def in_descs(k, b):
        ci = wid + k * (NC * NS)
        return [
            pltpu.make_async_copy(edge_hbm.at[ci], cnbuf[b], sem_in[b]),
            pltpu.make_async_copy(energy_hbm.at[ci], ebuf[b], sem_in[b]),
        ]

    def sc_descs(b):
        return [pltpu.make_async_copy(vbuf[b], acc.at[sidx[b]], sem_sc[b])]

    def compute(b):
        def row_body(r, carry):
            ebase = r * 128
            for t in range(8):
                o = t * LANES
                cvec = cnbuf[b][2 * r, pl.ds(o, LANES)]
                nvec = cnbuf[b][2 * r + 1, pl.ds(o, LANES)]
                cs = plsc.load_gather(species, [cvec])
                ns = plsc.load_gather(species, [nvec])
                scale = plsc.load_gather(table, [cs * NUM_TYPES + ns])
                vbuf[b][pl.ds(ebase + o, LANES)] = (
                    ebuf[b][r, pl.ds(o, LANES)] * scale)
                sidx[b][pl.ds(ebase + o, LANES)] = cvec
            return carry

        lax.fori_loop(0, ROWS, row_body, None)

    pltpu.sync_copy(species_hbm, species)
    pltpu.sync_copy(table_hbm, table)

    zero = jnp.zeros((LANES,), jnp.float32)
    for i in range(ZCHUNK // LANES):
        zbuf[pl.ds(i * LANES, LANES)] = zero
    base = sid * SLICE
    for k in range(4):
        pltpu.sync_copy(zbuf, acc.at[pl.ds(base + k * ZCHUNK, ZCHUNK)])
    plsc.subcore_barrier()

    n_chunks = jnp.where(wid < EXTRA_TILES, BASE_CHUNKS + 1, BASE_CHUNKS)

    for d in in_descs(0, 0):
        d.start()
    for d in in_descs(1, 1):
        d.start()

    def pair_body(p, _):
        for b in range(2):
            k = 2 * p + b

            @pl.when(k < n_chunks)
            def _process():
                for d in in_descs(k, b):
                    d.wait()

                @pl.when(k >= 2)
                def _drain():
                    for d in sc_descs(b):
                        d.wait()

                compute(b)
                for d in sc_descs(b):
                    d.start(add=True)

                @pl.when(k + 2 < n_chunks)
                def _prefetch():
                    for d in in_descs(k + 2, b):
                        d.start()

        return _

    lax.fori_loop(0, N_PAIRS, pair_body, None)

    for b in range(2):
        for d in sc_descs(b):
            d.wait()

    plsc.subcore_barrier()
    pltpu.sync_copy(acc.at[pl.ds(base, SLICE)],
                    part_hbm.at[cid, pl.ds(base, SLICE)])


_sc_kernel = pl.kernel(
    _sc_body,
    out_type=jax.ShapeDtypeStruct((NC, N_PAD), jnp.float32),
    mesh=plsc.VectorSubcoreMesh(core_axis_name="c", subcore_axis_name="s",
                                num_cores=NC, num_subcores=NS),
    compiler_params=pltpu.CompilerParams(needs_layout_passes=False),
    scratch_types=[
        pltpu.VMEM((N_NODES,), jnp.int32),
        pltpu.VMEM((NUM_TYPES * NUM_TYPES,), jnp.float32),
        pltpu.VMEM((2 * ROWS, 128), jnp.int32),
        pltpu.VMEM((ROWS, 128), jnp.float32),
        pltpu.VMEM((2 * ROWS, 128), jnp.int32),
        pltpu.VMEM((ROWS, 128), jnp.float32),
        pltpu.VMEM((CHUNK,), jnp.float32),
        pltpu.VMEM((CHUNK,), jnp.int32),
        pltpu.VMEM((CHUNK,), jnp.float32),
        pltpu.VMEM((CHUNK,), jnp.int32),
        pltpu.VMEM((ZCHUNK,), jnp.float32),
        pltpu.VMEM_SHARED((N_PAD,), jnp.float32),
        pltpu.SemaphoreType.DMA,
        pltpu.SemaphoreType.DMA,
        pltpu.SemaphoreType.DMA,
        pltpu.SemaphoreType.DMA,
    ],
)


def _combine_body(p_ref, o_ref):
    o_ref[...] = (p_ref[0] + p_ref[1]) * FACTOR


_combine = pl.pallas_call(
    _combine_body,
    out_shape=jax.ShapeDtypeStruct((N_PAD // 128, 128), jnp.float32),
)


def kernel(edge_energy, per_edge_scales, edge_index, atom_types):
    edges = (edge_index.reshape(2, N_EDGES // 128, 128)
             .transpose(1, 0, 2)
             .reshape(N_CHUNKS, 2 * ROWS, 128))
    energy = edge_energy.reshape(N_CHUNKS, ROWS, 128)
    species = atom_types.reshape(N_NODES)
    table = per_edge_scales.reshape(NUM_TYPES * NUM_TYPES)

    part = _sc_kernel(edges, energy, species, table)
    out = _combine(part.reshape(NC, N_PAD // 128, 128))
    return out.reshape(N_PAD)[:N_NODES].reshape(N_NODES, 1)


# speedup vs baseline: 1678.5859x; 2.2702x over previous
import math

import jax
import jax.numpy as jnp
from jax import lax
from jax.experimental import pallas as pl
from jax.experimental.pallas import tpu as pltpu
from jax.experimental.pallas import tpu_sc as plsc

N_NODES = 100000
N_EDGES = 6400000
NUM_TYPES = 16
FACTOR = 1.0 / math.sqrt(64.0)

NC = 2
NS = 16
LANES = 16

CHUNK = 2048
ROWS = CHUNK // 128
N_CHUNKS = N_EDGES // CHUNK
BASE_CHUNKS = N_CHUNKS // (NC * NS)
EXTRA_TILES = N_CHUNKS % (NC * NS)

N_PAD = 100352
SLICE = N_PAD // NS
ZCHUNK = SLICE // 4


N_PAIRS = (N_CHUNKS // (NC * NS) + 2) // 2


def _sc_body(edge_hbm, energy_hbm, species_hbm, table_hbm,
             part_hbm, species, table, cnbuf0, ebuf0, cnbuf1,
             ebuf1, vbuf0, sidx0, vbuf1, sidx1, zbuf, acc, sem_in0, sem_in1,
             sem_sc0, sem_sc1):
    cid = lax.axis_index("c")
    sid = lax.axis_index("s")
    wid = sid * NC + cid
    cnbuf = (cnbuf0, cnbuf1)
    ebuf = (ebuf0, ebuf1)
    vbuf = (vbuf0, vbuf1)
    sidx = (sidx0, sidx1)
    sem_in = (sem_in0, sem_in1)
    sem_sc = (sem_sc0, sem_sc1)

    def in_descs(k, b):
        ci = wid + k * (NC * NS)
        return [
            pltpu.make_async_copy(edge_hbm.at[ci], cnbuf[b], sem_in[b]),
            pltpu.make_async_copy(energy_hbm.at[ci], ebuf[b], sem_in[b]),
        ]

    def sc_descs(b):
        return [pltpu.make_async_copy(vbuf[b], acc.at[sidx[b]], sem_sc[b])]

    def compute(b):
        @plsc.parallel_loop(0, ROWS, unroll=2)
        def row_body(r):
            ebase = r * 128
            for t in range(8):
                o = t * LANES
                cvec = cnbuf[b][2 * r, pl.ds(o, LANES)]
                nvec = cnbuf[b][2 * r + 1, pl.ds(o, LANES)]
                cs = plsc.load_gather(species, [cvec])
                ns = plsc.load_gather(species, [nvec])
                scale = plsc.load_gather(table, [cs * NUM_TYPES + ns])
                vbuf[b][pl.ds(ebase + o, LANES)] = (
                    ebuf[b][r, pl.ds(o, LANES)] * scale)
                sidx[b][pl.ds(ebase + o, LANES)] = cvec

    pltpu.sync_copy(species_hbm, species)
    pltpu.sync_copy(table_hbm, table)

    zero = jnp.zeros((LANES,), jnp.float32)
    for i in range(ZCHUNK // LANES):
        zbuf[pl.ds(i * LANES, LANES)] = zero
    base = sid * SLICE
    for k in range(4):
        pltpu.sync_copy(zbuf, acc.at[pl.ds(base + k * ZCHUNK, ZCHUNK)])
    plsc.subcore_barrier()

    n_chunks = jnp.where(wid < EXTRA_TILES, BASE_CHUNKS + 1, BASE_CHUNKS)

    for d in in_descs(0, 0):
        d.start()
    for d in in_descs(1, 1):
        d.start()

    def pair_body(p, _):
        for b in range(2):
            k = 2 * p + b

            @pl.when(k < n_chunks)
            def _process():
                for d in in_descs(k, b):
                    d.wait()

                @pl.when(k >= 2)
                def _drain():
                    for d in sc_descs(b):
                        d.wait()

                compute(b)
                for d in sc_descs(b):
                    d.start(add=True)

                @pl.when(k + 2 < n_chunks)
                def _prefetch():
                    for d in in_descs(k + 2, b):
                        d.start()

        return _

    lax.fori_loop(0, N_PAIRS, pair_body, None)

    for b in range(2):
        for d in sc_descs(b):
            d.wait()

    plsc.subcore_barrier()
    pltpu.sync_copy(acc.at[pl.ds(base, SLICE)],
                    part_hbm.at[cid, pl.ds(base, SLICE)])


_sc_kernel = pl.kernel(
    _sc_body,
    out_type=jax.ShapeDtypeStruct((NC, N_PAD), jnp.float32),
    mesh=plsc.VectorSubcoreMesh(core_axis_name="c", subcore_axis_name="s",
                                num_cores=NC, num_subcores=NS),
    compiler_params=pltpu.CompilerParams(needs_layout_passes=False),
    scratch_types=[
        pltpu.VMEM((N_NODES,), jnp.int32),
        pltpu.VMEM((NUM_TYPES * NUM_TYPES,), jnp.float32),
        pltpu.VMEM((2 * ROWS, 128), jnp.int32),
        pltpu.VMEM((ROWS, 128), jnp.float32),
        pltpu.VMEM((2 * ROWS, 128), jnp.int32),
        pltpu.VMEM((ROWS, 128), jnp.float32),
        pltpu.VMEM((CHUNK,), jnp.float32),
        pltpu.VMEM((CHUNK,), jnp.int32),
        pltpu.VMEM((CHUNK,), jnp.float32),
        pltpu.VMEM((CHUNK,), jnp.int32),
        pltpu.VMEM((ZCHUNK,), jnp.float32),
        pltpu.VMEM_SHARED((N_PAD,), jnp.float32),
        pltpu.SemaphoreType.DMA,
        pltpu.SemaphoreType.DMA,
        pltpu.SemaphoreType.DMA,
        pltpu.SemaphoreType.DMA,
    ],
)


def _combine_body(p_ref, o_ref):
    o_ref[...] = (p_ref[0] + p_ref[1]) * FACTOR


_combine = pl.pallas_call(
    _combine_body,
    out_shape=jax.ShapeDtypeStruct((N_PAD // 128, 128), jnp.float32),
)


def kernel(edge_energy, per_edge_scales, edge_index, atom_types):
    edges = (edge_index.reshape(2, N_EDGES // 128, 128)
             .transpose(1, 0, 2)
             .reshape(N_CHUNKS, 2 * ROWS, 128))
    energy = edge_energy.reshape(N_CHUNKS, ROWS, 128)
    species = atom_types.reshape(N_NODES)
    table = per_edge_scales.reshape(NUM_TYPES * NUM_TYPES)

    part = _sc_kernel(edges, energy, species, table)
    out = _combine(part.reshape(NC, N_PAD // 128, 128))
    return out.reshape(N_PAD)[:N_NODES].reshape(N_NODES, 1)


# speedup vs baseline: 1715.3033x; 1.0219x over previous
import math

import jax
import jax.numpy as jnp
from jax import lax
from jax.experimental import pallas as pl
from jax.experimental.pallas import tpu as pltpu
from jax.experimental.pallas import tpu_sc as plsc

N_NODES = 100000
N_EDGES = 6400000
NUM_TYPES = 16
FACTOR = 1.0 / math.sqrt(64.0)

NC = 2
NS = 16
LANES = 16

CHUNK = 2048
ROWS = CHUNK // 128
N_CHUNKS = N_EDGES // CHUNK
BASE_CHUNKS = N_CHUNKS // (NC * NS)
EXTRA_TILES = N_CHUNKS % (NC * NS)

N_PAD = 100352
SLICE = N_PAD // NS
ZCHUNK = SLICE // 4


N_PAIRS = (N_CHUNKS // (NC * NS) + 2) // 2


def _sc_body(edge_hbm, energy_hbm, species_hbm, table_hbm,
             part_hbm, species, table, cnbuf0, ebuf0, cnbuf1,
             ebuf1, vbuf0, sidx0, vbuf1, sidx1, zbuf, acc, sem_in0, sem_in1,
             sem_sc0, sem_sc1):
    cid = lax.axis_index("c")
    sid = lax.axis_index("s")
    wid = sid * NC + cid
    cnbuf = (cnbuf0, cnbuf1)
    ebuf = (ebuf0, ebuf1)
    vbuf = (vbuf0, vbuf1)
    sidx = (sidx0, sidx1)
    sem_in = (sem_in0, sem_in1)
    sem_sc = (sem_sc0, sem_sc1)

    def in_descs(k, b):
        ci = wid + k * (NC * NS)
        return [
            pltpu.make_async_copy(edge_hbm.at[ci], cnbuf[b], sem_in[b]),
            pltpu.make_async_copy(energy_hbm.at[ci], ebuf[b], sem_in[b]),
        ]

    def sc_descs(b):
        return [pltpu.make_async_copy(vbuf[b], acc.at[sidx[b]], sem_sc[b])]

    def compute(b):
        @plsc.parallel_loop(0, ROWS, unroll=4)
        def row_body(r):
            ebase = r * 128
            for t in range(8):
                o = t * LANES
                cvec = cnbuf[b][2 * r, pl.ds(o, LANES)]
                nvec = cnbuf[b][2 * r + 1, pl.ds(o, LANES)]
                cs = plsc.load_gather(species, [cvec])
                ns = plsc.load_gather(species, [nvec])
                scale = plsc.load_gather(table, [cs * NUM_TYPES + ns])
                vbuf[b][pl.ds(ebase + o, LANES)] = (
                    ebuf[b][r, pl.ds(o, LANES)] * scale)
                sidx[b][pl.ds(ebase + o, LANES)] = cvec

    pltpu.sync_copy(species_hbm, species)
    pltpu.sync_copy(table_hbm, table)

    zero = jnp.zeros((LANES,), jnp.float32)
    for i in range(ZCHUNK // LANES):
        zbuf[pl.ds(i * LANES, LANES)] = zero
    base = sid * SLICE
    for k in range(4):
        pltpu.sync_copy(zbuf, acc.at[pl.ds(base + k * ZCHUNK, ZCHUNK)])
    plsc.subcore_barrier()

    n_chunks = jnp.where(wid < EXTRA_TILES, BASE_CHUNKS + 1, BASE_CHUNKS)

    for d in in_descs(0, 0):
        d.start()
    for d in in_descs(1, 1):
        d.start()

    def pair_body(p, _):
        for b in range(2):
            k = 2 * p + b

            @pl.when(k < n_chunks)
            def _process():
                for d in in_descs(k, b):
                    d.wait()

                @pl.when(k >= 2)
                def _drain():
                    for d in sc_descs(b):
                        d.wait()

                compute(b)
                for d in sc_descs(b):
                    d.start(add=True)

                @pl.when(k + 2 < n_chunks)
                def _prefetch():
                    for d in in_descs(k + 2, b):
                        d.start()

        return _

    lax.fori_loop(0, N_PAIRS, pair_body, None)

    for b in range(2):
        for d in sc_descs(b):
            d.wait()

    plsc.subcore_barrier()
    pltpu.sync_copy(acc.at[pl.ds(base, SLICE)],
                    part_hbm.at[cid, pl.ds(base, SLICE)])


_sc_kernel = pl.kernel(
    _sc_body,
    out_type=jax.ShapeDtypeStruct((NC, N_PAD), jnp.float32),
    mesh=plsc.VectorSubcoreMesh(core_axis_name="c", subcore_axis_name="s",
                                num_cores=NC, num_subcores=NS),
    compiler_params=pltpu.CompilerParams(needs_layout_passes=False),
    scratch_types=[
        pltpu.VMEM((N_NODES,), jnp.int32),
        pltpu.VMEM((NUM_TYPES * NUM_TYPES,), jnp.float32),
        pltpu.VMEM((2 * ROWS, 128), jnp.int32),
        pltpu.VMEM((ROWS, 128), jnp.float32),
        pltpu.VMEM((2 * ROWS, 128), jnp.int32),
        pltpu.VMEM((ROWS, 128), jnp.float32),
        pltpu.VMEM((CHUNK,), jnp.float32),
        pltpu.VMEM((CHUNK,), jnp.int32),
        pltpu.VMEM((CHUNK,), jnp.float32),
        pltpu.VMEM((CHUNK,), jnp.int32),
        pltpu.VMEM((ZCHUNK,), jnp.float32),
        pltpu.VMEM_SHARED((N_PAD,), jnp.float32),
        pltpu.SemaphoreType.DMA,
        pltpu.SemaphoreType.DMA,
        pltpu.SemaphoreType.DMA,
        pltpu.SemaphoreType.DMA,
    ],
)


def _combine_body(p_ref, o_ref):
    o_ref[...] = (p_ref[0] + p_ref[1]) * FACTOR


_combine = pl.pallas_call(
    _combine_body,
    out_shape=jax.ShapeDtypeStruct((N_PAD // 128, 128), jnp.float32),
)


def kernel(edge_energy, per_edge_scales, edge_index, atom_types):
    edges = (edge_index.reshape(2, N_EDGES // 128, 128)
             .transpose(1, 0, 2)
             .reshape(N_CHUNKS, 2 * ROWS, 128))
    energy = edge_energy.reshape(N_CHUNKS, ROWS, 128)
    species = atom_types.reshape(N_NODES)
    table = per_edge_scales.reshape(NUM_TYPES * NUM_TYPES)

    part = _sc_kernel(edges, energy, species, table)
    out = _combine(part.reshape(NC, N_PAD // 128, 128))
    return out.reshape(N_PAD)[:N_NODES].reshape(N_NODES, 1)
